# baseline (speedup 1.0000x reference)
import functools

import jax
import jax.numpy as jnp
from jax import lax
from jax.experimental import pallas as pl
from jax.experimental.pallas import tpu as pltpu
from jax.experimental.pallas import tpu_sc as plsc

NUM_HID = 64
SEQ = 200
NC, NS, L = 2, 16, 16
NW = NC * NS
CHUNK = 256
POS_ROWS = 456


def _pos_body(out_ref):
    half = NUM_HID // 2
    r = lax.broadcasted_iota(jnp.int32, (POS_ROWS, NUM_HID), 0)
    pos = (r % SEQ).astype(jnp.float32)
    j = lax.broadcasted_iota(jnp.int32, (POS_ROWS, NUM_HID), 1)
    jj = jnp.where(j < half, j, j - half).astype(jnp.float32) / float(half)
    rate = jnp.exp(jj * (-jnp.log(10000.0)))
    ang = pos * rate
    out_ref[...] = jnp.where(j < half, jnp.sin(ang), jnp.cos(ang))


_pos_table = pl.pallas_call(
    _pos_body,
    out_shape=jax.ShapeDtypeStruct((POS_ROWS, NUM_HID), jnp.float32),
)


def _make_sc_kernel(total_rows):
    rows_per_w = total_rows // NW
    n_chunks = rows_per_w // CHUNK

    @functools.partial(
        pl.kernel,
        out_type=jax.ShapeDtypeStruct((total_rows, NUM_HID), jnp.float32),
        mesh=plsc.VectorSubcoreMesh(core_axis_name="c", subcore_axis_name="s"),
        scratch_types=[
            pltpu.VMEM((2, 128), jnp.int32),
            pltpu.VMEM((CHUNK, NUM_HID), jnp.float32),
            pltpu.VMEM((POS_ROWS * NUM_HID,), jnp.float32),
            pltpu.SemaphoreType.DMA,
        ],
        compiler_params=pltpu.CompilerParams(use_tc_tiling_on_sc=False),
    )
    def sc_kernel(x_hbm, table_hbm, pos_hbm, out_hbm, idx_v, rows_v, pos_v, sem):
        wid = lax.axis_index("s") * NC + lax.axis_index("c")
        base = wid * rows_per_w
        pltpu.sync_copy(pos_hbm, pos_v)

        @pl.loop(0, n_chunks)
        def _chunk(c):
            off = base + c * CHUNK
            pltpu.sync_copy(x_hbm.at[pl.ds(off, 128)], idx_v.at[0])
            pltpu.sync_copy(x_hbm.at[pl.ds(off + 128, 128)], idx_v.at[1])
            cp0 = pltpu.async_copy(
                table_hbm.at[idx_v.at[0]], rows_v.at[pl.ds(0, 128)], sem)
            cp1 = pltpu.async_copy(
                table_hbm.at[idx_v.at[1]], rows_v.at[pl.ds(128, 128)], sem)
            cp0.wait()
            cp1.wait()
            p_off = lax.rem(c * CHUNK, SEQ) * NUM_HID

            @pl.loop(0, CHUNK)
            def _row(r):
                prow = p_off + r * NUM_HID
                for k in range(NUM_HID // L):
                    pv = pos_v[pl.ds(prow + k * L, L)]
                    rows_v[r, pl.ds(k * L, L)] = rows_v[r, pl.ds(k * L, L)] + pv

            pltpu.sync_copy(rows_v, out_hbm.at[pl.ds(off, CHUNK)])

    return sc_kernel


def kernel(x, table):
    b, s = x.shape
    total = b * s
    pos_flat = _pos_table().reshape(-1)
    x_flat = x.reshape(-1)
    out = _make_sc_kernel(total)(x_flat, table, pos_flat)
    return out.reshape(b, s, NUM_HID)


# speedup vs baseline: 1.1954x; 1.1954x over previous
import functools

import jax
import jax.numpy as jnp
from jax import lax
from jax.experimental import pallas as pl
from jax.experimental.pallas import tpu as pltpu
from jax.experimental.pallas import tpu_sc as plsc

NUM_HID = 64
SEQ = 200
NC, NS, L = 2, 16, 16
NW = NC * NS
CHUNK = 256
NBUF = 4
POS_ROWS = 456


def _pos_body(out_ref):
    half = NUM_HID // 2
    r = lax.broadcasted_iota(jnp.int32, (POS_ROWS, NUM_HID), 0)
    pos = (r % SEQ).astype(jnp.float32)
    j = lax.broadcasted_iota(jnp.int32, (POS_ROWS, NUM_HID), 1)
    jj = jnp.where(j < half, j, j - half).astype(jnp.float32) / float(half)
    rate = jnp.exp(jj * (-jnp.log(10000.0)))
    ang = pos * rate
    out_ref[...] = jnp.where(j < half, jnp.sin(ang), jnp.cos(ang))


_pos_table = pl.pallas_call(
    _pos_body,
    out_shape=jax.ShapeDtypeStruct((POS_ROWS, NUM_HID), jnp.float32),
)


def _make_sc_kernel(total_rows):
    rows_per_w = total_rows // NW
    n_chunks = rows_per_w // CHUNK

    @functools.partial(
        pl.kernel,
        out_type=jax.ShapeDtypeStruct((total_rows, NUM_HID), jnp.float32),
        mesh=plsc.VectorSubcoreMesh(core_axis_name="c", subcore_axis_name="s"),
        scratch_types=(
            [pltpu.VMEM((2, 128), jnp.int32) for _ in range(NBUF)]
            + [pltpu.VMEM((CHUNK, NUM_HID), jnp.float32) for _ in range(NBUF)]
            + [pltpu.VMEM((POS_ROWS * NUM_HID,), jnp.float32)]
            + [pltpu.SemaphoreType.DMA for _ in range(2 * NBUF)]
        ),
        compiler_params=pltpu.CompilerParams(use_tc_tiling_on_sc=False),
    )
    def sc_kernel(x_hbm, table_hbm, pos_hbm, out_hbm, *scratch):
        idx_v = scratch[:NBUF]
        rows_v = scratch[NBUF:2 * NBUF]
        pos_v = scratch[2 * NBUF]
        gsem = scratch[2 * NBUF + 1:2 * NBUF + 1 + NBUF]
        osem = scratch[2 * NBUF + 1 + NBUF:]

        wid = lax.axis_index("s") * NC + lax.axis_index("c")
        base = wid * rows_per_w
        pltpu.sync_copy(pos_hbm, pos_v)

        def load_idx(b, c):
            off = base + c * CHUNK
            pltpu.sync_copy(x_hbm.at[pl.ds(off, 128)], idx_v[b].at[0])
            pltpu.sync_copy(x_hbm.at[pl.ds(off + 128, 128)], idx_v[b].at[1])

        def fire_gather(b):
            pltpu.async_copy(
                table_hbm.at[idx_v[b].at[0]], rows_v[b].at[pl.ds(0, 128)],
                gsem[b])
            pltpu.async_copy(
                table_hbm.at[idx_v[b].at[1]], rows_v[b].at[pl.ds(128, 128)],
                gsem[b])

        def wait_gather(b):
            pltpu.make_async_copy(
                table_hbm.at[idx_v[b].at[0]], rows_v[b].at[pl.ds(0, 128)],
                gsem[b]).wait()
            pltpu.make_async_copy(
                table_hbm.at[idx_v[b].at[1]], rows_v[b].at[pl.ds(128, 128)],
                gsem[b]).wait()

        def out_desc(b, c):
            return pltpu.make_async_copy(
                rows_v[b], out_hbm.at[pl.ds(base + c * CHUNK, CHUNK)], osem[b])

        for b in range(NBUF):
            load_idx(b, b)
            fire_gather(b)

        @pl.loop(0, n_chunks, step=NBUF)
        def _round(g):
            for b in range(NBUF):
                c = g + b
                wait_gather(b)
                p_off = lax.rem(c * CHUNK, SEQ) * NUM_HID

                @pl.loop(0, CHUNK, unroll=8)
                def _row(r):
                    prow = p_off + r * NUM_HID
                    for k in range(NUM_HID // L):
                        pv = pos_v[pl.ds(prow + k * L, L)]
                        plsc.addupdate(rows_v[b].at[r, pl.ds(k * L, L)], pv)

                out_desc(b, c).start()
                nc = c + NBUF

                @pl.when(nc < n_chunks)
                def _next():
                    load_idx(b, nc)
                    out_desc(b, c).wait()
                    fire_gather(b)

        for b in range(NBUF):
            out_desc(b, n_chunks - NBUF + b).wait()

    return sc_kernel


def kernel(x, table):
    b, s = x.shape
    total = b * s
    pos_flat = _pos_table().reshape(-1)
    x_flat = x.reshape(-1)
    out = _make_sc_kernel(total)(x_flat, table, pos_flat)
    return out.reshape(b, s, NUM_HID)


# speedup vs baseline: 1.2207x; 1.0212x over previous
import functools

import jax
import jax.numpy as jnp
from jax import lax
from jax.experimental import pallas as pl
from jax.experimental.pallas import tpu as pltpu
from jax.experimental.pallas import tpu_sc as plsc

NUM_HID = 64
PAIR_HID = 128
SEQ = 200
NC, NS, L = 2, 16, 16
NW = NC * NS
CHUNK = 128
NBUF = 2
POS_ROWS = 328


def _pos_body(out_ref):
    half = NUM_HID // 2
    r = lax.broadcasted_iota(jnp.int32, (POS_ROWS, NUM_HID), 0)
    pos = (r % SEQ).astype(jnp.float32)
    j = lax.broadcasted_iota(jnp.int32, (POS_ROWS, NUM_HID), 1)
    jj = jnp.where(j < half, j, j - half).astype(jnp.float32) / float(half)
    rate = jnp.exp(jj * (-jnp.log(10000.0)))
    ang = pos * rate
    out_ref[...] = jnp.where(j < half, jnp.sin(ang), jnp.cos(ang))


_pos_table = pl.pallas_call(
    _pos_body,
    out_shape=jax.ShapeDtypeStruct((POS_ROWS, NUM_HID), jnp.float32),
)


def _make_sc_kernel(total_rows):
    rows_per_w = total_rows // NW
    n_chunks = rows_per_w // CHUNK

    @functools.partial(
        pl.kernel,
        out_type=jax.ShapeDtypeStruct((total_rows, NUM_HID), jnp.float32),
        mesh=plsc.VectorSubcoreMesh(core_axis_name="c", subcore_axis_name="s"),
        scratch_types=(
            [pltpu.VMEM((CHUNK,), jnp.int32) for _ in range(NBUF)]
            + [pltpu.VMEM((CHUNK, PAIR_HID), jnp.float32) for _ in range(NBUF)]
            + [pltpu.VMEM((CHUNK, NUM_HID), jnp.float32) for _ in range(NBUF)]
            + [pltpu.VMEM((CHUNK,), jnp.int32) for _ in range(NBUF)]
            + [pltpu.VMEM((POS_ROWS, NUM_HID), jnp.float32)]
            + [pltpu.SemaphoreType.DMA for _ in range(2 * NBUF)]
        ),
        compiler_params=pltpu.CompilerParams(use_tc_tiling_on_sc=True,
                                             disable_bounds_checks=True),
    )
    def sc_kernel(x_hbm, table2_hbm, pos_hbm, out_hbm, *scratch):
        pidx_v = scratch[:NBUF]
        pair_v = scratch[NBUF:2 * NBUF]
        rows_v = scratch[2 * NBUF:3 * NBUF]
        raw_s = scratch[3 * NBUF:4 * NBUF]
        pos_v = scratch[4 * NBUF]
        gsem = scratch[4 * NBUF + 1:4 * NBUF + 1 + NBUF]
        osem = scratch[4 * NBUF + 1 + NBUF:4 * NBUF + 1 + 2 * NBUF]

        wid = lax.axis_index("s") * NC + lax.axis_index("c")
        base = wid * rows_per_w
        pltpu.sync_copy(pos_hbm, pos_v)

        def fire_chunk(b, c):
            off = base + c * CHUNK
            pltpu.sync_copy(x_hbm.at[pl.ds(off, CHUNK)], raw_s[b])

            @pl.loop(0, CHUNK // L)
            def _half(i):
                pidx_v[b][pl.ds(i * L, L)] = lax.shift_right_logical(
                    raw_s[b][pl.ds(i * L, L)], 1)

            pltpu.async_copy(table2_hbm.at[pidx_v[b]], pair_v[b], gsem[b])

        def wait_chunk(b):
            pltpu.make_async_copy(
                table2_hbm.at[pidx_v[b]], pair_v[b], gsem[b]).wait()

        def out_desc(b, c):
            return pltpu.make_async_copy(
                rows_v[b], out_hbm.at[pl.ds(base + c * CHUNK, CHUNK)], osem[b])

        for b in range(NBUF):
            fire_chunk(b, b)

        @pl.loop(0, n_chunks, step=NBUF)
        def _round(g):
            for b in range(NBUF):
                c = g + b
                wait_chunk(b)
                p0 = lax.rem(c * CHUNK, SEQ)

                @pl.loop(0, CHUNK // L)
                def _grp(g):
                    parv = lax.shift_left(
                        lax.bitwise_and(raw_s[b][pl.ds(g * L, L)], 1), 6)
                    for l in range(L):
                        r = g * L + l
                        cbase = parv[l]
                        for k in range(NUM_HID // L):
                            pv = pair_v[b][r, pl.ds(cbase + k * L, L)]
                            po = pos_v[p0 + r, pl.ds(k * L, L)]
                            rows_v[b][r, pl.ds(k * L, L)] = pv + po

                out_desc(b, c).start()
                nc = c + NBUF

                @pl.when(nc < n_chunks)
                def _next():
                    out_desc(b, c).wait()
                    fire_chunk(b, nc)

        for b in range(NBUF):
            out_desc(b, n_chunks - NBUF + b).wait()

    return sc_kernel


def kernel(x, table):
    b, s = x.shape
    total = b * s
    v = table.shape[0]
    pos = _pos_table()
    x_flat = x.reshape(-1)
    table2 = table.reshape(v // 2, 2 * NUM_HID)
    out = _make_sc_kernel(total)(x_flat, table2, pos)
    return out.reshape(b, s, NUM_HID)


# speedup vs baseline: 1.3074x; 1.0710x over previous
import functools

import jax
import jax.numpy as jnp
from jax import lax
from jax.experimental import pallas as pl
from jax.experimental.pallas import tpu as pltpu
from jax.experimental.pallas import tpu_sc as plsc

NUM_HID = 64
PAIR_HID = 128
SEQ = 200
NC, NS, L = 2, 16, 16
NW = NC * NS
CHUNK = 128
NBUF = 2
POS_ROWS = 328


def _pos_body(out_ref):
    half = NUM_HID // 2
    r = lax.broadcasted_iota(jnp.int32, (POS_ROWS, NUM_HID), 0)
    pos = (r % SEQ).astype(jnp.float32)
    j = lax.broadcasted_iota(jnp.int32, (POS_ROWS, NUM_HID), 1)
    jj = jnp.where(j < half, j, j - half).astype(jnp.float32) / float(half)
    rate = jnp.exp(jj * (-jnp.log(10000.0)))
    ang = pos * rate
    out_ref[...] = jnp.where(j < half, jnp.sin(ang), jnp.cos(ang))


_pos_table = pl.pallas_call(
    _pos_body,
    out_shape=jax.ShapeDtypeStruct((POS_ROWS, NUM_HID), jnp.float32),
)


def _make_sc_kernel(total_rows):
    rows_per_w = total_rows // NW
    n_chunks = rows_per_w // CHUNK

    @functools.partial(
        pl.kernel,
        out_type=jax.ShapeDtypeStruct((total_rows, NUM_HID), jnp.float32),
        mesh=plsc.VectorSubcoreMesh(core_axis_name="c", subcore_axis_name="s"),
        scratch_types=(
            [pltpu.VMEM((CHUNK,), jnp.int32) for _ in range(NBUF)]
            + [pltpu.VMEM((CHUNK,), jnp.int32) for _ in range(NBUF)]
            + [pltpu.VMEM((CHUNK, PAIR_HID), jnp.float32) for _ in range(NBUF)]
            + [pltpu.VMEM((CHUNK, NUM_HID), jnp.float32) for _ in range(NBUF)]
            + [pltpu.VMEM((POS_ROWS, NUM_HID), jnp.float32)]
            + [pltpu.SemaphoreType.DMA for _ in range(3 * NBUF)]
        ),
        compiler_params=pltpu.CompilerParams(use_tc_tiling_on_sc=True,
                                             disable_bounds_checks=True),
    )
    def sc_kernel(x2_hbm, par_hbm, table2_hbm, pos_hbm, out_hbm, *scratch):
        pidx_v = scratch[:NBUF]
        par_v = scratch[NBUF:2 * NBUF]
        pair_v = scratch[2 * NBUF:3 * NBUF]
        rows_v = scratch[3 * NBUF:4 * NBUF]
        pos_v = scratch[4 * NBUF]
        isem = scratch[4 * NBUF + 1:4 * NBUF + 1 + NBUF]
        gsem = scratch[4 * NBUF + 1 + NBUF:4 * NBUF + 1 + 2 * NBUF]
        osem = scratch[4 * NBUF + 1 + 2 * NBUF:]

        wid = lax.axis_index("s") * NC + lax.axis_index("c")
        base = wid * rows_per_w
        pltpu.sync_copy(pos_hbm, pos_v)

        def idx_start(b, c):
            off = base + c * CHUNK
            pltpu.async_copy(x2_hbm.at[pl.ds(off, CHUNK)], pidx_v[b], isem[b])
            pltpu.async_copy(par_hbm.at[pl.ds(off, CHUNK)], par_v[b], isem[b])

        def idx_wait(b):
            pltpu.make_async_copy(
                x2_hbm.at[pl.ds(0, CHUNK)], pidx_v[b], isem[b]).wait()
            pltpu.make_async_copy(
                par_hbm.at[pl.ds(0, CHUNK)], par_v[b], isem[b]).wait()

        def gather_start(b):
            pltpu.async_copy(table2_hbm.at[pidx_v[b]], pair_v[b], gsem[b])

        def gather_wait(b):
            pltpu.make_async_copy(
                table2_hbm.at[pidx_v[b]], pair_v[b], gsem[b]).wait()

        def out_desc(b, c):
            return pltpu.make_async_copy(
                rows_v[b], out_hbm.at[pl.ds(base + c * CHUNK, CHUNK)], osem[b])

        for b in range(NBUF):
            idx_start(b, b)
        for b in range(NBUF):
            idx_wait(b)
            gather_start(b)

        @pl.loop(0, n_chunks, step=NBUF)
        def _round(g):
            for b in range(NBUF):
                c = g + b
                gather_wait(b)
                nc = c + NBUF

                @pl.when(nc < n_chunks)
                def _pref():
                    idx_start(b, nc)

                p0 = lax.rem(c * CHUNK, SEQ)

                @pl.loop(0, CHUNK // L)
                def _grp(g16):
                    parv = par_v[b][pl.ds(g16 * L, L)]
                    for l in range(L):
                        r = g16 * L + l
                        cbase = parv[l]
                        for k in range(NUM_HID // L):
                            pv = pair_v[b][r, pl.ds(cbase + k * L, L)]
                            po = pos_v[p0 + r, pl.ds(k * L, L)]
                            rows_v[b][r, pl.ds(k * L, L)] = pv + po

                out_desc(b, c).start()

                @pl.when(nc < n_chunks)
                def _next():
                    out_desc(b, c).wait()
                    idx_wait(b)
                    gather_start(b)

        for b in range(NBUF):
            out_desc(b, n_chunks - NBUF + b).wait()

    return sc_kernel


def kernel(x, table):
    b, s = x.shape
    total = b * s
    v = table.shape[0]
    pos = _pos_table()
    x_flat = x.reshape(-1)
    x2 = lax.shift_right_logical(x_flat, 1)
    par = lax.shift_left(lax.bitwise_and(x_flat, 1), 6)
    table2 = table.reshape(v // 2, 2 * NUM_HID)
    out = _make_sc_kernel(total)(x2, par, table2, pos)
    return out.reshape(b, s, NUM_HID)
